# baseline (speedup 1.0000x reference)
import functools

import jax
import jax.numpy as jnp
from jax import lax
from jax.experimental import pallas as pl
from jax.experimental.pallas import tpu as pltpu
from jax.experimental.pallas import tpu_sc as plsc

NC = 2
NS = 16
LANES = 16
CH = 128


def _zero_share(acc, zsrc, row0, rows_per_tile):
    off = 0
    while off < rows_per_tile:
        sz = min(CH, rows_per_tile - off)
        pltpu.sync_copy(zsrc.at[pl.ds(0, sz)], acc.at[pl.ds(row0 + off, sz)])
        off += sz


def _load_dst_group(dst_hbm, dgrp, tb, g, dsem):
    for jj in range(GP):
        off = pl.multiple_of(tb + (g * GP + jj) * CH, CH)
        pltpu.async_copy(dst_hbm.at[pl.ds(off, CH)], dgrp.at[jj], dsem)
    for jj in range(GP):
        pltpu.make_async_copy(dst_hbm.at[pl.ds(0, CH)], dgrp.at[jj],
                              dsem).wait()


def _deg_body(cpt, n, dst_hbm, ew_hbm, out_hbm, acc, eall, dgrp,
              ebuf0, ebuf1, ssem0, ssem1, dsem):
    c = lax.axis_index("c")
    s = lax.axis_index("s")
    wid = s * NC + c
    tb = pl.multiple_of(wid * cpt * CH, CH)
    rows_per_tile = n // NS
    ebuf = (ebuf0, ebuf1)
    ssem = (ssem0, ssem1)

    @plsc.parallel_loop(0, CH, unroll=2)
    def zfill(j):
        ebuf0[j, :] = jnp.zeros((LANES,), jnp.float32)
    _zero_share(acc, ebuf0, s * rows_per_tile, rows_per_tile)
    plsc.subcore_barrier()

    pltpu.sync_copy(ew_hbm.at[pl.ds(tb, cpt * CH)], eall)

    def group(g, _):
        _load_dst_group(dst_hbm, dgrp, tb, g, dsem)

        def pair(pp, _):
            scat = []
            for b in (0, 1):
                jj = 2 * pp + b
                ibase = (g * GP + jj) * CH

                @plsc.parallel_loop(0, CH, unroll=2)
                def fill(j, _b=b, _ibase=ibase):
                    w = plsc.load_gather(
                        eall, [jnp.full((LANES,), _ibase + j, jnp.int32)])
                    ebuf[_b][j, :] = w
                scat.append(pltpu.async_copy(ebuf[b], acc.at[dgrp.at[jj]],
                                             ssem[b], add=True))
            for b in (0, 1):
                scat[b].wait()
            return 0
        lax.fori_loop(0, GP // 2, pair, 0)
        return 0
    lax.fori_loop(0, cpt // GP, group, 0)
    plsc.subcore_barrier()

    pltpu.sync_copy(acc.at[pl.ds(s * rows_per_tile, rows_per_tile)],
                    out_hbm.at[c, pl.ds(s * rows_per_tile, rows_per_tile)])


GP = 16


def _agg_body(cpt, n, d, g_hbm, src_hbm, dst_hbm, ew_hbm, out_hbm,
              acc, sall, dgrp, egrp, rows0, rows1, gsem0, gsem1,
              ssem0, ssem1, dsem):
    c = lax.axis_index("c")
    s = lax.axis_index("s")
    wid = s * NC + c
    tb = pl.multiple_of(wid * cpt * CH, CH)
    rows_per_tile = n // NS
    nsub = d // LANES
    rows = (rows0, rows1)
    gsem = (gsem0, gsem1)
    ssem = (ssem0, ssem1)

    @plsc.parallel_loop(0, CH, unroll=2)
    def zfill(j):
        for r in range(nsub):
            rows0[j, pl.ds(r * LANES, LANES)] = jnp.zeros((LANES,), jnp.float32)
    _zero_share(acc, rows0, s * rows_per_tile, rows_per_tile)
    plsc.subcore_barrier()

    pltpu.sync_copy(src_hbm.at[pl.ds(tb, cpt * CH)], sall)
    pltpu.async_copy(g_hbm.at[sall.at[pl.ds(0, CH)]], rows0, gsem0)
    pltpu.async_copy(g_hbm.at[sall.at[pl.ds(CH, CH)]], rows1, gsem1)

    def group(g, _):
        _load_dst_group(dst_hbm, dgrp, tb, g, dsem)
        _load_dst_group(ew_hbm, egrp, tb, g, dsem)

        def pair(pp, _):
            scat = []
            for b in (0, 1):
                jj = 2 * pp + b
                pltpu.make_async_copy(g_hbm.at[sall.at[pl.ds(0, CH)]],
                                      rows[b], gsem[b]).wait()

                @plsc.parallel_loop(0, CH, unroll=4)
                def scale(jx, _b=b, _jj=jj):
                    w = plsc.load_gather(
                        egrp, [jnp.full((LANES,), _jj, jnp.int32),
                               jnp.full((LANES,), jx, jnp.int32)])
                    for r in range(nsub):
                        sl = pl.ds(r * LANES, LANES)
                        rows[_b][jx, sl] = rows[_b][jx, sl] * w
                scat.append(pltpu.async_copy(rows[b], acc.at[dgrp.at[jj]],
                                             ssem[b], add=True))
            for b in (0, 1):
                i = g * GP + 2 * pp + b
                nxt = jnp.where(i + 2 < cpt, i + 2, 0)
                noff = pl.multiple_of(nxt * CH, CH)
                scat[b].wait()
                pltpu.async_copy(g_hbm.at[sall.at[pl.ds(noff, CH)]],
                                 rows[b], gsem[b])
            return 0
        lax.fori_loop(0, GP // 2, pair, 0)
        return 0
    lax.fori_loop(0, cpt // GP, group, 0)
    pltpu.make_async_copy(g_hbm.at[sall.at[pl.ds(0, CH)]], rows0, gsem0).wait()
    pltpu.make_async_copy(g_hbm.at[sall.at[pl.ds(0, CH)]], rows1, gsem1).wait()
    plsc.subcore_barrier()

    pltpu.sync_copy(acc.at[pl.ds(s * rows_per_tile, rows_per_tile)],
                    out_hbm.at[c, pl.ds(s * rows_per_tile, rows_per_tile)])


def _sc_deg(dstp, ewp, n, cpt):
    mesh = plsc.VectorSubcoreMesh(core_axis_name="c", subcore_axis_name="s",
                                  num_cores=NC, num_subcores=NS)
    return pl.kernel(
        functools.partial(_deg_body, cpt, n),
        out_type=jax.ShapeDtypeStruct((NC, n, LANES), jnp.float32),
        mesh=mesh,
        compiler_params=pltpu.CompilerParams(needs_layout_passes=False,
                                             use_tc_tiling_on_sc=False),
        scratch_types=[
            pltpu.VMEM_SHARED((n, LANES), jnp.float32),
            pltpu.VMEM((cpt * CH,), jnp.float32),
            pltpu.VMEM((GP, CH), jnp.int32),
            pltpu.VMEM((CH, LANES), jnp.float32),
            pltpu.VMEM((CH, LANES), jnp.float32),
            pltpu.SemaphoreType.DMA,
            pltpu.SemaphoreType.DMA,
            pltpu.SemaphoreType.DMA,
        ],
    )(dstp, ewp)


def _sc_agg(g, srcp, dstp, ewp, n, d, cpt):
    mesh = plsc.VectorSubcoreMesh(core_axis_name="c", subcore_axis_name="s",
                                  num_cores=NC, num_subcores=NS)
    return pl.kernel(
        functools.partial(_agg_body, cpt, n, d),
        out_type=jax.ShapeDtypeStruct((NC, n, d), jnp.float32),
        mesh=mesh,
        compiler_params=pltpu.CompilerParams(needs_layout_passes=False),
        scratch_types=[
            pltpu.VMEM_SHARED((n, d), jnp.float32),
            pltpu.VMEM((cpt * CH,), jnp.int32),
            pltpu.VMEM((GP, CH), jnp.int32),
            pltpu.VMEM((GP, CH), jnp.float32),
            pltpu.VMEM((CH, d), jnp.float32),
            pltpu.VMEM((CH, d), jnp.float32),
            pltpu.SemaphoreType.DMA,
            pltpu.SemaphoreType.DMA,
            pltpu.SemaphoreType.DMA,
            pltpu.SemaphoreType.DMA,
            pltpu.SemaphoreType.DMA,
        ],
    )(g, srcp, dstp, ewp)


def _dinv(deg_ref):
    deg = deg_ref[0, :, 0:1] + deg_ref[1, :, 0:1] + 1.0
    return jnp.where(deg > 0, lax.rsqrt(deg), 0.0)


def _mm_scale_body(deg_ref, x_ref, w_ref, o_ref):
    dinv = _dinv(deg_ref)
    h = jnp.dot(x_ref[...], w_ref[...], preferred_element_type=jnp.float32)
    o_ref[...] = h * dinv


def _mid_body(deg_ref, s_ref, g_ref, b_ref, w_ref, o_ref):
    dinv = _dinv(deg_ref)
    t = dinv * (s_ref[0] + s_ref[1] + g_ref[...]) + b_ref[...]
    z = jnp.maximum(t, 0.0)
    o_ref[...] = jnp.dot(z, w_ref[...], preferred_element_type=jnp.float32) * dinv


def _final_body(deg_ref, s_ref, g_ref, b_ref, o_ref):
    dinv = _dinv(deg_ref)
    o_ref[...] = dinv * (s_ref[0] + s_ref[1] + g_ref[...]) + b_ref[...]


def _tc_call(body, inputs, n, d, rb, out_rows=None):
    grid = (n // rb,)
    specs = []
    for a in inputs:
        if a.ndim == 3:
            specs.append(pl.BlockSpec((2, rb, a.shape[2]), lambda i: (0, i, 0)))
        elif a.shape[0] >= n:
            specs.append(pl.BlockSpec((rb, a.shape[1]), lambda i: (i, 0)))
        else:
            specs.append(pl.BlockSpec(a.shape, lambda i: (0,) * a.ndim))
    return pl.pallas_call(
        body,
        grid=grid,
        in_specs=specs,
        out_specs=pl.BlockSpec((rb, d), lambda i: (i, 0)),
        out_shape=jax.ShapeDtypeStruct((out_rows or n, d), jnp.float32),
    )(*inputs)


def kernel(x, edge_index, edge_attr, W1, b1, W2, b2):
    n, d_in = x.shape
    d_hid = W1.shape[1]
    d_out = W2.shape[1]
    e = edge_attr.shape[0]

    np_ = -(-n // 128) * 128

    nw = NC * NS
    cpt = -(-e // (nw * CH * GP)) * GP
    ep = nw * CH * cpt
    pad = ep - e
    pad_idx = (jnp.arange(pad, dtype=jnp.int32) % n)
    srcp = jnp.concatenate([edge_index[0], pad_idx])
    dstp = jnp.concatenate([edge_index[1], pad_idx])
    ewp = jnp.concatenate([edge_attr, jnp.zeros((pad,), edge_attr.dtype)])

    degp = _sc_deg(dstp, ewp, np_, cpt)

    rb = 1000
    b1r = b1.reshape(1, d_hid)
    b2r = b2.reshape(1, d_out)

    g1 = _tc_call(_mm_scale_body, [degp, x, W1], n, d_hid, rb, np_)
    s1 = _sc_agg(g1, srcp, dstp, ewp, np_, d_hid, cpt)
    g2 = _tc_call(_mid_body, [degp, s1, g1, b1r, W2], n, d_hid, rb, np_)
    s2 = _sc_agg(g2, srcp, dstp, ewp, np_, d_out, cpt)
    out = _tc_call(_final_body, [degp, s2, g2, b2r], n, d_out, rb)
    return out


# speedup vs baseline: 1.0172x; 1.0172x over previous
import functools

import jax
import jax.numpy as jnp
from jax import lax
from jax.experimental import pallas as pl
from jax.experimental.pallas import tpu as pltpu
from jax.experimental.pallas import tpu_sc as plsc

NC = 2
NS = 16
LANES = 16
CH = 128


def _zero_share(acc, zsrc, row0, rows_per_tile):
    off = 0
    while off < rows_per_tile:
        sz = min(CH, rows_per_tile - off)
        pltpu.sync_copy(zsrc.at[pl.ds(0, sz)], acc.at[pl.ds(row0 + off, sz)])
        off += sz


def _load_dst_group(dst_hbm, dgrp, tb, g, dsem):
    for jj in range(GP):
        off = pl.multiple_of(tb + (g * GP + jj) * CH, CH)
        pltpu.async_copy(dst_hbm.at[pl.ds(off, CH)], dgrp.at[jj], dsem)
    for jj in range(GP):
        pltpu.make_async_copy(dst_hbm.at[pl.ds(0, CH)], dgrp.at[jj],
                              dsem).wait()


def _deg_body(cpt, n, dst_hbm, ew_hbm, out_hbm, acc, eall, dgrp,
              ebuf0, ebuf1, ssem0, ssem1, dsem):
    c = lax.axis_index("c")
    s = lax.axis_index("s")
    wid = s * NC + c
    tb = pl.multiple_of(wid * cpt * CH, CH)
    rows_per_tile = n // NS
    ebuf = (ebuf0, ebuf1)
    ssem = (ssem0, ssem1)

    @plsc.parallel_loop(0, CH, unroll=2)
    def zfill(j):
        ebuf0[j, :] = jnp.zeros((LANES,), jnp.float32)
    _zero_share(acc, ebuf0, s * rows_per_tile, rows_per_tile)
    plsc.subcore_barrier()

    pltpu.sync_copy(ew_hbm.at[pl.ds(tb, cpt * CH)], eall)

    def group(g, _):
        _load_dst_group(dst_hbm, dgrp, tb, g, dsem)

        def pair(pp, _):
            scat = []
            for b in (0, 1):
                jj = 2 * pp + b
                ibase = (g * GP + jj) * CH

                @plsc.parallel_loop(0, CH, unroll=2)
                def fill(j, _b=b, _ibase=ibase):
                    w = plsc.load_gather(
                        eall, [jnp.full((LANES,), _ibase + j, jnp.int32)])
                    ebuf[_b][j, :] = w
                scat.append(pltpu.async_copy(ebuf[b], acc.at[dgrp.at[jj]],
                                             ssem[b], add=True))
            for b in (0, 1):
                scat[b].wait()
            return 0
        lax.fori_loop(0, GP // 2, pair, 0)
        return 0
    lax.fori_loop(0, cpt // GP, group, 0)
    plsc.subcore_barrier()

    pltpu.sync_copy(acc.at[pl.ds(s * rows_per_tile, rows_per_tile)],
                    out_hbm.at[c, pl.ds(s * rows_per_tile, rows_per_tile)])


GP = 16


def _agg_body(cpt, n, d, g_hbm, src_hbm, dst_hbm, ew_hbm, out_hbm,
              acc, sall, dgrp, egrp, rows0, rows1, gsem0, gsem1,
              ssem0, ssem1, dsem):
    c = lax.axis_index("c")
    s = lax.axis_index("s")
    wid = s * NC + c
    tb = pl.multiple_of(wid * cpt * CH, CH)
    rows_per_tile = n // NS
    nsub = d // LANES
    rows = (rows0, rows1)
    gsem = (gsem0, gsem1)
    ssem = (ssem0, ssem1)

    @plsc.parallel_loop(0, CH, unroll=2)
    def zfill(j):
        for r in range(nsub):
            rows0[j, pl.ds(r * LANES, LANES)] = jnp.zeros((LANES,), jnp.float32)
    _zero_share(acc, rows0, s * rows_per_tile, rows_per_tile)
    plsc.subcore_barrier()

    pltpu.sync_copy(src_hbm.at[pl.ds(tb, cpt * CH)], sall)
    pltpu.async_copy(g_hbm.at[sall.at[pl.ds(0, CH)]], rows0, gsem0)
    pltpu.async_copy(g_hbm.at[sall.at[pl.ds(CH, CH)]], rows1, gsem1)

    def group(g, _):
        _load_dst_group(dst_hbm, dgrp, tb, g, dsem)
        _load_dst_group(ew_hbm, egrp, tb, g, dsem)

        def pair(pp, _):
            scat = []
            for b in (0, 1):
                jj = 2 * pp + b
                pltpu.make_async_copy(g_hbm.at[sall.at[pl.ds(0, CH)]],
                                      rows[b], gsem[b]).wait()

                @plsc.parallel_loop(0, CH, unroll=4)
                def scale(jx, _b=b, _jj=jj):
                    w = plsc.load_gather(
                        egrp, [jnp.full((LANES,), _jj, jnp.int32),
                               jnp.full((LANES,), jx, jnp.int32)])
                    for r in range(nsub):
                        sl = pl.ds(r * LANES, LANES)
                        rows[_b][jx, sl] = rows[_b][jx, sl] * w
                scat.append(pltpu.async_copy(rows[b], acc.at[dgrp.at[jj]],
                                             ssem[b], add=True))
            for b in (0, 1):
                i = g * GP + 2 * pp + b
                nxt = jnp.where(i + 2 < cpt, i + 2, 0)
                noff = pl.multiple_of(nxt * CH, CH)
                scat[b].wait()
                pltpu.async_copy(g_hbm.at[sall.at[pl.ds(noff, CH)]],
                                 rows[b], gsem[b])
            return 0
        lax.fori_loop(0, GP // 2, pair, 0)
        return 0
    lax.fori_loop(0, cpt // GP, group, 0)
    pltpu.make_async_copy(g_hbm.at[sall.at[pl.ds(0, CH)]], rows0, gsem0).wait()
    pltpu.make_async_copy(g_hbm.at[sall.at[pl.ds(0, CH)]], rows1, gsem1).wait()
    plsc.subcore_barrier()

    pltpu.sync_copy(acc.at[pl.ds(s * rows_per_tile, rows_per_tile)],
                    out_hbm.at[c, pl.ds(s * rows_per_tile, rows_per_tile)])


def _sc_deg(dstp, ewp, n, cpt):
    mesh = plsc.VectorSubcoreMesh(core_axis_name="c", subcore_axis_name="s",
                                  num_cores=NC, num_subcores=NS)
    return pl.kernel(
        functools.partial(_deg_body, cpt, n),
        out_type=jax.ShapeDtypeStruct((NC, n, LANES), jnp.float32),
        mesh=mesh,
        compiler_params=pltpu.CompilerParams(needs_layout_passes=False,
                                             use_tc_tiling_on_sc=False),
        scratch_types=[
            pltpu.VMEM_SHARED((n, LANES), jnp.float32),
            pltpu.VMEM((cpt * CH,), jnp.float32),
            pltpu.VMEM((GP, CH), jnp.int32),
            pltpu.VMEM((CH, LANES), jnp.float32),
            pltpu.VMEM((CH, LANES), jnp.float32),
            pltpu.SemaphoreType.DMA,
            pltpu.SemaphoreType.DMA,
            pltpu.SemaphoreType.DMA,
        ],
    )(dstp, ewp)


def _sc_agg(g, srcp, dstp, ewp, n, d, cpt):
    mesh = plsc.VectorSubcoreMesh(core_axis_name="c", subcore_axis_name="s",
                                  num_cores=NC, num_subcores=NS)
    return pl.kernel(
        functools.partial(_agg_body, cpt, n, d),
        out_type=jax.ShapeDtypeStruct((NC, n, d), jnp.float32),
        mesh=mesh,
        compiler_params=pltpu.CompilerParams(needs_layout_passes=False),
        scratch_types=[
            pltpu.VMEM_SHARED((n, d), jnp.float32),
            pltpu.VMEM((cpt * CH,), jnp.int32),
            pltpu.VMEM((GP, CH), jnp.int32),
            pltpu.VMEM((GP, CH), jnp.float32),
            pltpu.VMEM((CH, d), jnp.float32),
            pltpu.VMEM((CH, d), jnp.float32),
            pltpu.SemaphoreType.DMA,
            pltpu.SemaphoreType.DMA,
            pltpu.SemaphoreType.DMA,
            pltpu.SemaphoreType.DMA,
            pltpu.SemaphoreType.DMA,
        ],
    )(g, srcp, dstp, ewp)


def _dinv(deg_ref):
    deg = deg_ref[0, :, 0:1] + deg_ref[1, :, 0:1] + 1.0
    return jnp.where(deg > 0, lax.rsqrt(deg), 0.0)


def _mm_scale_body(deg_ref, x_ref, w_ref, o_ref):
    dinv = _dinv(deg_ref)
    h = jnp.dot(x_ref[...], w_ref[...], preferred_element_type=jnp.float32)
    o_ref[...] = h * dinv


def _mid_body(deg_ref, s_ref, g_ref, b_ref, w_ref, o_ref):
    dinv = _dinv(deg_ref)
    t = dinv * (s_ref[0] + s_ref[1] + g_ref[...]) + b_ref[...]
    z = jnp.maximum(t, 0.0)
    o_ref[...] = jnp.dot(z, w_ref[...], preferred_element_type=jnp.float32) * dinv


def _final_body(deg_ref, s_ref, g_ref, b_ref, o_ref):
    dinv = _dinv(deg_ref)
    o_ref[...] = dinv * (s_ref[0] + s_ref[1] + g_ref[...]) + b_ref[...]


def _tc_call(body, inputs, n, d, rb, out_rows=None):
    grid = (n // rb,)
    specs = []
    for a in inputs:
        if a.ndim == 3:
            specs.append(pl.BlockSpec((2, rb, a.shape[2]), lambda i: (0, i, 0)))
        elif a.shape[0] >= n:
            specs.append(pl.BlockSpec((rb, a.shape[1]), lambda i: (i, 0)))
        else:
            specs.append(pl.BlockSpec(a.shape, lambda i: (0,) * a.ndim))
    return pl.pallas_call(
        body,
        grid=grid,
        in_specs=specs,
        out_specs=pl.BlockSpec((rb, d), lambda i: (i, 0)),
        out_shape=jax.ShapeDtypeStruct((out_rows or n, d), jnp.float32),
    )(*inputs)


def kernel(x, edge_index, edge_attr, W1, b1, W2, b2):
    n, d_in = x.shape
    d_hid = W1.shape[1]
    d_out = W2.shape[1]
    e = edge_attr.shape[0]

    np_ = -(-n // 128) * 128

    nw = NC * NS
    cpt = -(-e // (nw * CH * GP)) * GP
    ep = nw * CH * cpt
    pad = ep - e
    pad_idx = (jnp.arange(pad, dtype=jnp.int32) % n)
    srcp = jnp.concatenate([edge_index[0], pad_idx])
    dstp = jnp.concatenate([edge_index[1], pad_idx])
    ewp = jnp.concatenate([edge_attr, jnp.zeros((pad,), edge_attr.dtype)])

    degp = _sc_deg(dstp, ewp, np_, cpt)

    rb = 2000
    b1r = b1.reshape(1, d_hid)
    b2r = b2.reshape(1, d_out)

    g1 = _tc_call(_mm_scale_body, [degp, x, W1], n, d_hid, rb, np_)
    s1 = _sc_agg(g1, srcp, dstp, ewp, np_, d_hid, cpt)
    g2 = _tc_call(_mid_body, [degp, s1, g1, b1r, W2], n, d_hid, rb, np_)
    s2 = _sc_agg(g2, srcp, dstp, ewp, np_, d_out, cpt)
    out = _tc_call(_final_body, [degp, s2, g2, b2r], n, d_out, rb)
    return out
